# baseline (speedup 1.0000x reference)
import jax
import jax.numpy as jnp
from jax.experimental import pallas as pl
from jax.experimental.pallas import tpu as pltpu

_BLOCK_ROWS = 1024


def _body(id_ref, emb_ref, tok_ref, out_ref):
    idx = id_ref[0]
    row = emb_ref[pl.ds(idx, 1), :]
    out_ref[...] = tok_ref[...] + row


def kernel(tokens, embed_weight, type_id):
    B, N, D = tokens.shape
    rows = B * N
    flat = tokens.reshape(rows, D)
    tid = jnp.asarray(type_id, jnp.int32).reshape(1)
    grid = rows // _BLOCK_ROWS
    out = pl.pallas_call(
        _body,
        grid=(grid,),
        in_specs=[
            pl.BlockSpec(memory_space=pltpu.SMEM),
            pl.BlockSpec(embed_weight.shape, lambda i: (0, 0)),
            pl.BlockSpec((_BLOCK_ROWS, D), lambda i: (i, 0)),
        ],
        out_specs=pl.BlockSpec((_BLOCK_ROWS, D), lambda i: (i, 0)),
        out_shape=jax.ShapeDtypeStruct((rows, D), tokens.dtype),
    )(tid, embed_weight, flat)
    return out.reshape(B, N, D)
